# baseline (speedup 1.0000x reference)
import functools

import jax
import jax.numpy as jnp
from jax import lax
from jax.experimental import pallas as pl
from jax.experimental.pallas import tpu as pltpu
from jax.experimental.pallas import tpu_sc as plsc

N_NODES = 10000
N_EDGES_RAW = 320000
E_TOTAL = N_EDGES_RAW
CHUNK = 128
N_SUBCORES = 16
ROWS_PER_TILE = 160
EDGE_ROWS = ROWS_PER_TILE * N_SUBCORES
E_PAD = EDGE_ROWS * CHUNK
ACC_ROWS = 10112
DUMP_ROW = N_NODES


@functools.lru_cache(maxsize=None)
def _make_sc_agg(dhalf: int, npairs: int):
    mesh = plsc.VectorSubcoreMesh(core_axis_name="c", subcore_axis_name="s")

    NB = 5
    N_GROUPS = ROWS_PER_TILE // NB

    def body(*refs):
        tabs = refs[:2 * npairs]
        src_hbm, dst_hbm = refs[2 * npairs:2 * npairs + 2]
        outs = refs[2 * npairs + 2:4 * npairs + 2]
        src_idx, dst_idx, bufs, acc, gsem, ssem = refs[4 * npairs + 2:]

        c = lax.axis_index("c")
        s = lax.axis_index("s")
        rows_per_tile_acc = ACC_ROWS // N_SUBCORES

        zeros16 = jnp.zeros((16,), jnp.float32)

        def zero_acc():
            def zero_row(i, carry):
                for k in range(dhalf // 16):
                    bufs[0, i, pl.ds(k * 16, 16)] = zeros16
                return carry

            lax.fori_loop(0, CHUNK, zero_row, 0)
            done = 0
            while done < rows_per_tile_acc:
                cnt = min(CHUNK, rows_per_tile_acc - done)
                pltpu.sync_copy(
                    bufs.at[0, pl.ds(0, cnt)],
                    acc.at[pl.ds(s * rows_per_tile_acc + done, cnt)])
                done += cnt

        zero_acc()

        pltpu.sync_copy(src_hbm.at[pl.ds(s * ROWS_PER_TILE, ROWS_PER_TILE)],
                        src_idx)
        pltpu.sync_copy(dst_hbm.at[pl.ds(s * ROWS_PER_TILE, ROWS_PER_TILE)],
                        dst_idx)

        def run(table):
            def fire_gather(b, j):
                pltpu.async_copy(table.at[src_idx.at[j]], bufs.at[b],
                                 gsem.at[b])

            for b in range(NB):
                fire_gather(b, b)

            def group(t, carry):
                base = t * NB
                for b in range(NB):
                    pltpu.make_async_copy(table.at[src_idx.at[base + b]],
                                          bufs.at[b], gsem.at[b]).wait()
                    pltpu.async_copy(bufs.at[b], acc.at[dst_idx.at[base + b]],
                                     ssem.at[b], add=True)

                @pl.when(t + 1 < N_GROUPS)
                def _():
                    for b in range(NB):
                        pltpu.make_async_copy(
                            bufs.at[b], acc.at[dst_idx.at[base + b]],
                            ssem.at[b]).wait()
                        fire_gather(b, base + NB + b)

                @pl.when(t + 1 >= N_GROUPS)
                def _():
                    for b in range(NB):
                        pltpu.make_async_copy(
                            bufs.at[b], acc.at[dst_idx.at[base + b]],
                            ssem.at[b]).wait()
                return carry

            lax.fori_loop(0, N_GROUPS, group, 0)

        for p in range(npairs):
            plsc.subcore_barrier()

            @pl.when(c == 0)
            def _():
                run(tabs[2 * p])

            @pl.when(c == 1)
            def _():
                run(tabs[2 * p + 1])

            plsc.subcore_barrier()

            @pl.when(c == 0)
            def _():
                pltpu.sync_copy(
                    acc.at[pl.ds(s * rows_per_tile_acc, rows_per_tile_acc)],
                    outs[2 * p].at[pl.ds(s * rows_per_tile_acc,
                                         rows_per_tile_acc)])

            @pl.when(c == 1)
            def _():
                pltpu.sync_copy(
                    acc.at[pl.ds(s * rows_per_tile_acc, rows_per_tile_acc)],
                    outs[2 * p + 1].at[pl.ds(s * rows_per_tile_acc,
                                             rows_per_tile_acc)])

            if p + 1 < npairs:
                zero_acc()

    return pl.kernel(
        body,
        out_type=tuple(
            jax.ShapeDtypeStruct((ACC_ROWS, dhalf), jnp.float32)
            for _ in range(2 * npairs)),
        mesh=mesh,
        compiler_params=pltpu.CompilerParams(use_tc_tiling_on_sc=False, needs_layout_passes=False),
        scratch_types=[
            pltpu.VMEM((ROWS_PER_TILE, CHUNK), jnp.int32),
            pltpu.VMEM((ROWS_PER_TILE, CHUNK), jnp.int32),
            pltpu.VMEM((NB, CHUNK, dhalf), jnp.float32),
            pltpu.VMEM_SHARED((ACC_ROWS, dhalf), jnp.float32),
            pltpu.SemaphoreType.DMA((NB,)),
            pltpu.SemaphoreType.DMA((NB,)),
        ],
    )


_DEG_ROWS_PER_TILE = EDGE_ROWS // 32


def _make_sc_deg():
    mesh = plsc.VectorSubcoreMesh(core_axis_name="c", subcore_axis_name="s")

    def body(dst_hbm, out, dst_v, partial):
        c = lax.axis_index("c")
        s = lax.axis_index("s")
        wid = s * 2 + c

        zeros16 = jnp.zeros((16,), jnp.float32)

        def zero16(i, carry):
            partial[pl.ds(i * 16, 16)] = zeros16
            return carry

        lax.fori_loop(0, ACC_ROWS // 16, zero16, 0)

        pltpu.sync_copy(
            dst_hbm.at[pl.ds(wid * _DEG_ROWS_PER_TILE, _DEG_ROWS_PER_TILE)],
            dst_v)

        ones16 = jnp.ones((16,), jnp.float32)

        def row(j, carry):
            for k in range(CHUNK // 16):
                idx = dst_v[j, pl.ds(k * 16, 16)]
                plsc.addupdate_scatter(partial, [idx], ones16)
            return carry

        lax.fori_loop(0, _DEG_ROWS_PER_TILE, row, 0)
        pltpu.sync_copy(partial, out.at[wid])

    return pl.kernel(
        body,
        out_type=jax.ShapeDtypeStruct((32, ACC_ROWS), jnp.float32),
        mesh=mesh,
        scratch_types=[
            pltpu.VMEM((_DEG_ROWS_PER_TILE, CHUNK), jnp.int32),
            pltpu.VMEM((ACC_ROWS,), jnp.float32),
        ],
        compiler_params=pltpu.CompilerParams(use_tc_tiling_on_sc=False, needs_layout_passes=False),
    )


_ROW_BLK = 2000
_GRID = (N_NODES // _ROW_BLK,)


def _row_spec(cols):
    return pl.BlockSpec((_ROW_BLK, cols), lambda i: (i, 0))


def _full_spec(r, c):
    return pl.BlockSpec((r, c), lambda i: (0, 0))


_QW = 64


def _quarter_outs(n):
    nq = n // _QW
    specs = tuple(_row_spec(_QW) for _ in range(nq))
    shapes = tuple(jax.ShapeDtypeStruct((N_NODES, _QW), jnp.float32)
                   for _ in range(nq))
    return nq, specs, shapes


def _tc_first(x, deg_parts):
    n = x.shape[1]
    nq, q_specs, q_shapes = _quarter_outs(n)

    def body(x_ref, deg_ref, *outs):
        deg = jnp.sum(deg_ref[0], axis=-1).reshape(_ROW_BLK, 1) + 1.0
        dis = lax.rsqrt(deg)
        xs = x_ref[...] * dis
        for q in range(nq):
            outs[q][...] = xs[:, q * _QW:(q + 1) * _QW]
        outs[nq][...] = dis

    return pl.pallas_call(
        body,
        grid=_GRID,
        in_specs=[_row_spec(x.shape[1]),
                  pl.BlockSpec((1, _ROW_BLK, 32), lambda i: (i, 0, 0))],
        out_specs=q_specs + (_row_spec(1),),
        out_shape=q_shapes + (jax.ShapeDtypeStruct((N_NODES, 1), jnp.float32),),
    )(x, deg_parts)


def _tc_mid(aggs, prevs, dis2d, b2d, w):
    n = w.shape[1]
    nq, q_specs, q_shapes = _quarter_outs(n)
    n_a = len(aggs)
    n_p = len(prevs)

    def body(*refs):
        a_refs = refs[:n_a]
        p_refs = refs[n_a:n_a + n_p]
        dis_ref, b_ref, w_ref = refs[n_a + n_p:n_a + n_p + 3]
        outs = refs[n_a + n_p + 3:]
        dis = dis_ref[...]
        g = jnp.concatenate([r[...] for r in a_refs], axis=1)
        p = jnp.concatenate([r[...] for r in p_refs], axis=1)
        h = jnp.maximum((g + p) * dis + b_ref[...], 0.0)
        hs = jnp.dot(h, w_ref[...], preferred_element_type=jnp.float32) * dis
        for q in range(nq):
            outs[q][...] = hs[:, q * _QW:(q + 1) * _QW]

    return pl.pallas_call(
        body,
        grid=_GRID,
        in_specs=[_row_spec(a.shape[1]) for a in aggs]
        + [_row_spec(p.shape[1]) for p in prevs]
        + [_row_spec(1), _full_spec(*b2d.shape), _full_spec(*w.shape)],
        out_specs=q_specs,
        out_shape=q_shapes,
    )(*aggs, *prevs, dis2d, b2d, w)


def _tc_mid2(aggs, prevs, dis2d, b1_2d, w1, w2):
    n = w2.shape[1]
    nq, q_specs, q_shapes = _quarter_outs(n)
    n_a = len(aggs)
    n_p = len(prevs)

    def body(*refs):
        a_refs = refs[:n_a]
        p_refs = refs[n_a:n_a + n_p]
        dis_ref, b1_ref, w1_ref, w2_ref = refs[n_a + n_p:n_a + n_p + 4]
        outs = refs[n_a + n_p + 4:]
        dis = dis_ref[...]
        g = jnp.concatenate([r[...] for r in a_refs], axis=1)
        p = jnp.concatenate([r[...] for r in p_refs], axis=1)
        t = (g + p) * dis
        h1 = jnp.maximum(
            jnp.dot(t, w1_ref[...], preferred_element_type=jnp.float32)
            + b1_ref[...], 0.0)
        hs2 = jnp.dot(h1, w2_ref[...],
                      preferred_element_type=jnp.float32) * dis
        for q in range(nq):
            outs[q][...] = hs2[:, q * _QW:(q + 1) * _QW]

    return pl.pallas_call(
        body,
        grid=_GRID,
        in_specs=[_row_spec(a.shape[1]) for a in aggs]
        + [_row_spec(p.shape[1]) for p in prevs]
        + [_row_spec(1), _full_spec(*b1_2d.shape), _full_spec(*w1.shape),
           _full_spec(*w2.shape)],
        out_specs=q_specs,
        out_shape=q_shapes,
    )(*aggs, *prevs, dis2d, b1_2d, w1, w2)


def _tc_last(aggs, prevs, dis2d, b2d):
    n_a = len(aggs)
    n_p = len(prevs)

    def body(*refs):
        a_refs = refs[:n_a]
        p_refs = refs[n_a:n_a + n_p]
        dis_ref, b_ref, out_ref = refs[n_a + n_p:]
        g = jnp.concatenate([r[...] for r in a_refs], axis=1)
        p = jnp.concatenate([r[...] for r in p_refs], axis=1)
        out_ref[...] = (g + p) * dis_ref[...] + b_ref[...]

    n = sum(a.shape[1] for a in aggs)
    return pl.pallas_call(
        body,
        grid=_GRID,
        in_specs=[_row_spec(a.shape[1]) for a in aggs]
        + [_row_spec(p.shape[1]) for p in prevs]
        + [_row_spec(1), _full_spec(*b2d.shape)],
        out_specs=_row_spec(n),
        out_shape=jax.ShapeDtypeStruct((N_NODES, n), jnp.float32),
    )(*aggs, *prevs, dis2d, b2d)


def kernel(x, edge_index, W1, b1, W2, b2, W3, b3):
    pad = E_PAD - E_TOTAL
    src = jnp.concatenate(
        [edge_index[0].astype(jnp.int32),
         jnp.zeros((pad,), jnp.int32)]).reshape(EDGE_ROWS, CHUNK)
    dst = jnp.concatenate(
        [edge_index[1].astype(jnp.int32),
         jnp.full((pad,), DUMP_ROW, jnp.int32)]).reshape(EDGE_ROWS, CHUNK)

    deg_parts = _make_sc_deg()(dst)
    deg_parts = deg_parts[:, :N_NODES].reshape(
        32, N_NODES // _ROW_BLK, _ROW_BLK).transpose(1, 2, 0)

    def aggregate(quarters):
        agg = _make_sc_agg(_QW, len(quarters) // 2)
        return list(agg(*quarters, src, dst))

    *xs, dis2d = _tc_first(x, deg_parts)
    a = aggregate(xs)

    hs = _tc_mid2(a, xs, dis2d, b1.reshape(1, -1), W1, W2)
    a = aggregate(hs)

    hs = _tc_mid(a, hs, dis2d, b2.reshape(1, -1), W3)
    a = aggregate(hs)

    return _tc_last(a, hs, dis2d, b3.reshape(1, -1))


# speedup vs baseline: 2.2640x; 2.2640x over previous
import functools

import jax
import jax.numpy as jnp
from jax import lax
from jax.experimental import pallas as pl
from jax.experimental.pallas import tpu as pltpu
from jax.experimental.pallas import tpu_sc as plsc

N_NODES = 10000
N_EDGES_RAW = 320000
E_TOTAL = N_EDGES_RAW
CHUNK = 128
N_SUBCORES = 16
ROWS_PER_TILE = 160
EDGE_ROWS = ROWS_PER_TILE * N_SUBCORES
E_PAD = EDGE_ROWS * CHUNK
ACC_ROWS = 10112
DUMP_ROW = N_NODES


@functools.lru_cache(maxsize=None)
def _make_sc_agg(dhalf: int, npairs: int):
    mesh = plsc.VectorSubcoreMesh(core_axis_name="c", subcore_axis_name="s")

    NB = 5
    N_GROUPS = ROWS_PER_TILE // NB

    def body(*refs):
        tabs = refs[:2 * npairs]
        src_hbm, dst_hbm = refs[2 * npairs:2 * npairs + 2]
        outs = refs[2 * npairs + 2:4 * npairs + 2]
        src_idx, dst_idx, bufs, acc, gsem, ssem = refs[4 * npairs + 2:]

        c = lax.axis_index("c")
        s = lax.axis_index("s")
        rows_per_tile_acc = ACC_ROWS // N_SUBCORES

        zeros16 = jnp.zeros((16,), jnp.float32)

        def zero_acc():
            def zero_row(i, carry):
                for k in range(dhalf // 16):
                    bufs[0, i, pl.ds(k * 16, 16)] = zeros16
                return carry

            lax.fori_loop(0, CHUNK, zero_row, 0)
            done = 0
            while done < rows_per_tile_acc:
                cnt = min(CHUNK, rows_per_tile_acc - done)
                pltpu.sync_copy(
                    bufs.at[0, pl.ds(0, cnt)],
                    acc.at[pl.ds(s * rows_per_tile_acc + done, cnt)])
                done += cnt

        zero_acc()

        pltpu.sync_copy(src_hbm.at[pl.ds(s * ROWS_PER_TILE, ROWS_PER_TILE)],
                        src_idx)
        pltpu.sync_copy(dst_hbm.at[pl.ds(s * ROWS_PER_TILE, ROWS_PER_TILE)],
                        dst_idx)

        def run(table):
            def fire_gather(b, j):
                pltpu.async_copy(table.at[src_idx.at[j]], bufs.at[b],
                                 gsem.at[b])

            for b in range(NB):
                fire_gather(b, b)

            def group(t, carry):
                base = t * NB
                for b in range(NB):
                    pltpu.make_async_copy(table.at[src_idx.at[base + b]],
                                          bufs.at[b], gsem.at[b]).wait()
                    pltpu.async_copy(bufs.at[b], acc.at[dst_idx.at[base + b]],
                                     ssem.at[b], add=True)

                @pl.when(t + 1 < N_GROUPS)
                def _():
                    for b in range(NB):
                        pltpu.make_async_copy(
                            bufs.at[b], acc.at[dst_idx.at[base + b]],
                            ssem.at[b]).wait()
                        fire_gather(b, base + NB + b)

                @pl.when(t + 1 >= N_GROUPS)
                def _():
                    for b in range(NB):
                        pltpu.make_async_copy(
                            bufs.at[b], acc.at[dst_idx.at[base + b]],
                            ssem.at[b]).wait()
                return carry

            lax.fori_loop(0, N_GROUPS, group, 0)

        for p in range(npairs):
            plsc.subcore_barrier()

            @pl.when(c == 0)
            def _():
                run(tabs[2 * p])

            @pl.when(c == 1)
            def _():
                run(tabs[2 * p + 1])

            plsc.subcore_barrier()

            @pl.when(c == 0)
            def _():
                pltpu.sync_copy(
                    acc.at[pl.ds(s * rows_per_tile_acc, rows_per_tile_acc)],
                    outs[2 * p].at[pl.ds(s * rows_per_tile_acc,
                                         rows_per_tile_acc)])

            @pl.when(c == 1)
            def _():
                pltpu.sync_copy(
                    acc.at[pl.ds(s * rows_per_tile_acc, rows_per_tile_acc)],
                    outs[2 * p + 1].at[pl.ds(s * rows_per_tile_acc,
                                             rows_per_tile_acc)])

            if p + 1 < npairs:
                zero_acc()

    return pl.kernel(
        body,
        out_type=tuple(
            jax.ShapeDtypeStruct((ACC_ROWS, dhalf), jnp.float32)
            for _ in range(2 * npairs)),
        mesh=mesh,
        compiler_params=pltpu.CompilerParams(use_tc_tiling_on_sc=False, needs_layout_passes=False),
        scratch_types=[
            pltpu.VMEM((ROWS_PER_TILE, CHUNK), jnp.int32),
            pltpu.VMEM((ROWS_PER_TILE, CHUNK), jnp.int32),
            pltpu.VMEM((NB, CHUNK, dhalf), jnp.float32),
            pltpu.VMEM_SHARED((ACC_ROWS, dhalf), jnp.float32),
            pltpu.SemaphoreType.DMA((NB,)),
            pltpu.SemaphoreType.DMA((NB,)),
        ],
    )


_DEG_ROWS_PER_TILE = EDGE_ROWS // 32


def _make_sc_deg():
    mesh = plsc.VectorSubcoreMesh(core_axis_name="c", subcore_axis_name="s")

    def body(dst_hbm, out, dst_v, partial):
        c = lax.axis_index("c")
        s = lax.axis_index("s")
        wid = s * 2 + c

        zeros16 = jnp.zeros((16,), jnp.float32)

        def zero16(i, carry):
            partial[pl.ds(i * 16, 16)] = zeros16
            return carry

        lax.fori_loop(0, ACC_ROWS // 16, zero16, 0)

        pltpu.sync_copy(
            dst_hbm.at[pl.ds(wid * _DEG_ROWS_PER_TILE, _DEG_ROWS_PER_TILE)],
            dst_v)

        ones16 = jnp.ones((16,), jnp.float32)

        def row(j, carry):
            for k in range(CHUNK // 16):
                idx = dst_v[j, pl.ds(k * 16, 16)]
                plsc.addupdate_scatter(partial, [idx], ones16)
            return carry

        lax.fori_loop(0, _DEG_ROWS_PER_TILE, row, 0)
        pltpu.sync_copy(partial, out.at[wid])

    return pl.kernel(
        body,
        out_type=jax.ShapeDtypeStruct((32, ACC_ROWS), jnp.float32),
        mesh=mesh,
        scratch_types=[
            pltpu.VMEM((_DEG_ROWS_PER_TILE, CHUNK), jnp.int32),
            pltpu.VMEM((ACC_ROWS,), jnp.float32),
        ],
        compiler_params=pltpu.CompilerParams(use_tc_tiling_on_sc=False, needs_layout_passes=False),
    )


_ROW_BLK = 2000
_GRID = (N_NODES // _ROW_BLK,)


def _row_spec(cols):
    return pl.BlockSpec((_ROW_BLK, cols), lambda i: (i, 0))


def _full_spec(r, c):
    return pl.BlockSpec((r, c), lambda i: (0, 0))


_QW = 64


def _quarter_outs(n):
    nq = n // _QW
    specs = tuple(_row_spec(_QW) for _ in range(nq))
    shapes = tuple(jax.ShapeDtypeStruct((N_NODES, _QW), jnp.float32)
                   for _ in range(nq))
    return nq, specs, shapes


def _tc_first(x, deg_parts):
    n = x.shape[1]
    nq, q_specs, q_shapes = _quarter_outs(n)

    def body(x_ref, deg_ref, *outs):
        deg = jnp.sum(deg_ref[0], axis=-1).reshape(_ROW_BLK, 1) + 1.0
        dis = lax.rsqrt(deg)
        xs = x_ref[...] * dis
        for q in range(nq):
            outs[q][...] = xs[:, q * _QW:(q + 1) * _QW]
        outs[nq][...] = dis

    return pl.pallas_call(
        body,
        grid=_GRID,
        in_specs=[_row_spec(x.shape[1]),
                  pl.BlockSpec((1, _ROW_BLK, 32), lambda i: (i, 0, 0))],
        out_specs=q_specs + (_row_spec(1),),
        out_shape=q_shapes + (jax.ShapeDtypeStruct((N_NODES, 1), jnp.float32),),
    )(x, deg_parts)


def _tc_mid(aggs, prevs, dis2d, b2d, w):
    n = w.shape[1]
    nq, q_specs, q_shapes = _quarter_outs(n)
    n_a = len(aggs)
    n_p = len(prevs)

    def body(*refs):
        a_refs = refs[:n_a]
        p_refs = refs[n_a:n_a + n_p]
        dis_ref, b_ref, w_ref = refs[n_a + n_p:n_a + n_p + 3]
        outs = refs[n_a + n_p + 3:]
        dis = dis_ref[...]
        g = jnp.concatenate([r[...] for r in a_refs], axis=1)
        p = jnp.concatenate([r[...] for r in p_refs], axis=1)
        h = jnp.maximum((g + p) * dis + b_ref[...], 0.0)
        hs = jnp.dot(h, w_ref[...], preferred_element_type=jnp.float32) * dis
        for q in range(nq):
            outs[q][...] = hs[:, q * _QW:(q + 1) * _QW]

    return pl.pallas_call(
        body,
        grid=_GRID,
        in_specs=[_row_spec(a.shape[1]) for a in aggs]
        + [_row_spec(p.shape[1]) for p in prevs]
        + [_row_spec(1), _full_spec(*b2d.shape), _full_spec(*w.shape)],
        out_specs=q_specs,
        out_shape=q_shapes,
    )(*aggs, *prevs, dis2d, b2d, w)


def _tc_mid2(aggs, prevs, dis2d, b1_2d, w1, w2):
    n = w2.shape[1]
    nq, q_specs, q_shapes = _quarter_outs(n)
    n_a = len(aggs)
    n_p = len(prevs)

    def body(*refs):
        a_refs = refs[:n_a]
        p_refs = refs[n_a:n_a + n_p]
        dis_ref, b1_ref, w1_ref, w2_ref = refs[n_a + n_p:n_a + n_p + 4]
        outs = refs[n_a + n_p + 4:]
        dis = dis_ref[...]
        g = jnp.concatenate([r[...] for r in a_refs], axis=1)
        p = jnp.concatenate([r[...] for r in p_refs], axis=1)
        t = (g + p) * dis
        h1 = jnp.maximum(
            jnp.dot(t, w1_ref[...], preferred_element_type=jnp.float32)
            + b1_ref[...], 0.0)
        hs2 = jnp.dot(h1, w2_ref[...],
                      preferred_element_type=jnp.float32) * dis
        for q in range(nq):
            outs[q][...] = hs2[:, q * _QW:(q + 1) * _QW]

    return pl.pallas_call(
        body,
        grid=_GRID,
        in_specs=[_row_spec(a.shape[1]) for a in aggs]
        + [_row_spec(p.shape[1]) for p in prevs]
        + [_row_spec(1), _full_spec(*b1_2d.shape), _full_spec(*w1.shape),
           _full_spec(*w2.shape)],
        out_specs=q_specs,
        out_shape=q_shapes,
    )(*aggs, *prevs, dis2d, b1_2d, w1, w2)


def _tc_last(aggs, prevs, dis2d, b2d):
    n_a = len(aggs)
    n_p = len(prevs)

    def body(*refs):
        a_refs = refs[:n_a]
        p_refs = refs[n_a:n_a + n_p]
        dis_ref, b_ref, out_ref = refs[n_a + n_p:]
        g = jnp.concatenate([r[...] for r in a_refs], axis=1)
        p = jnp.concatenate([r[...] for r in p_refs], axis=1)
        out_ref[...] = (g + p) * dis_ref[...] + b_ref[...]

    n = sum(a.shape[1] for a in aggs)
    return pl.pallas_call(
        body,
        grid=_GRID,
        in_specs=[_row_spec(a.shape[1]) for a in aggs]
        + [_row_spec(p.shape[1]) for p in prevs]
        + [_row_spec(1), _full_spec(*b2d.shape)],
        out_specs=_row_spec(n),
        out_shape=jax.ShapeDtypeStruct((N_NODES, n), jnp.float32),
    )(*aggs, *prevs, dis2d, b2d)


def kernel(x, edge_index, W1, b1, W2, b2, W3, b3):
    pad = E_PAD - E_TOTAL
    pad_ids = jnp.arange(pad, dtype=jnp.int32)
    src = jnp.concatenate(
        [edge_index[0].astype(jnp.int32),
         pad_ids % N_NODES]).reshape(EDGE_ROWS, CHUNK)
    dst = jnp.concatenate(
        [edge_index[1].astype(jnp.int32),
         DUMP_ROW + pad_ids % (ACC_ROWS - N_NODES)]).reshape(EDGE_ROWS, CHUNK)

    deg_parts = _make_sc_deg()(dst)
    deg_parts = deg_parts[:, :N_NODES].reshape(
        32, N_NODES // _ROW_BLK, _ROW_BLK).transpose(1, 2, 0)

    def aggregate(quarters):
        agg = _make_sc_agg(_QW, len(quarters) // 2)
        outs = agg(*quarters, src, dst)
        return [t[:N_NODES] for t in outs]

    *xs, dis2d = _tc_first(x, deg_parts)
    a = aggregate(xs)

    hs = _tc_mid2(a, xs, dis2d, b1.reshape(1, -1), W1, W2)
    a = aggregate(hs)

    hs = _tc_mid(a, hs, dis2d, b2.reshape(1, -1), W3)
    a = aggregate(hs)

    return _tc_last(a, hs, dis2d, b3.reshape(1, -1))
